# baseline (speedup 1.0000x reference)
import functools

import jax
import jax.numpy as jnp
from jax import lax
from jax.experimental import pallas as pl
from jax.experimental.pallas import tpu as pltpu
from jax.experimental.pallas import tpu_sc as plsc

N_NODES = 10000
N_EDGES = 320000
D = 128
HD = D // 2
NC, NS = 2, 16
CHUNK = 128
NCHUNK = 2560
CPT = NCHUNK // NS
E_PAD = NCHUNK * CHUNK
ACC_R = 10240
RPT = ACC_R // NS
ZCH = 128
DEGW = 16


@functools.cache
def _mesh():
  return plsc.VectorSubcoreMesh(
      core_axis_name="c", subcore_axis_name="s", num_cores=NC, num_subcores=NS)


def _gather_scatter_loop(h, src_v, dst_v, bufs, gsems, ssems, acc, deg_fn):
  def g(jj, b):
    pltpu.async_copy(h.at[src_v.at[jj]], bufs[b], gsems[b])

  def gwait(jj, b):
    pltpu.make_async_copy(h.at[src_v.at[jj]], bufs[b], gsems[b]).wait()

  def sc(jj, b):
    pltpu.async_copy(bufs[b], acc.at[dst_v.at[jj]], ssems[b], add=True)

  def scwait(jj, b):
    pltpu.make_async_copy(bufs[b], acc.at[dst_v.at[jj]], ssems[b]).wait()

  g(0, 0)
  g(1, 1)
  g(2, 2)
  gwait(0, 0)
  sc(0, 0)
  deg_fn(0, 0, True)
  g(3, 3)
  gwait(1, 1)
  sc(1, 1)
  deg_fn(1, 1, True)

  def body(i, carry):
    jj = 4 * i + 2
    for t in range(4):
      b = (2 + t) % 4
      bn = t % 4
      scwait(jj + t - 2, bn)
      g(jj + t + 2, bn)
      gwait(jj + t, b)
      sc(jj + t, b)
      deg_fn(jj + t, t % 2, False)
    return carry

  lax.fori_loop(0, (CPT - 4) // 4, body, 0)

  for jj in (CPT - 2, CPT - 1):
    gwait(jj, jj % 4)
    sc(jj, jj % 4)
    deg_fn(jj, jj % 2, False)
  for jj in range(CPT - 4, CPT):
    scwait(jj, jj % 4)


def _seg_sum_deg_body(h, srcr, dstr, z64, z16, ones, outp, degp,
                      src_v, dst_v, b0, b1, b2, b3, ones_v, acc, dacc,
                      g0, g1, g2, g3, s0, s1, s2, s3, dsem):
  c = lax.axis_index("c")
  s = lax.axis_index("s")
  for k in range(RPT // ZCH):
    pltpu.sync_copy(z64, acc.at[pl.ds(s * RPT + k * ZCH, ZCH)])
  pltpu.sync_copy(z16, dacc.at[pl.ds(s * RPT, RPT)])
  pltpu.sync_copy(ones, ones_v)
  pltpu.sync_copy(srcr.at[c, pl.ds(s * CPT, CPT)], src_v)
  pltpu.sync_copy(dstr.at[pl.ds(s * CPT, CPT)], dst_v)
  plsc.subcore_barrier()
  col = pl.ds(pl.multiple_of(c * HD, HD), HD)

  def deg(jj, par, first):
    @pl.when(c == par)
    def _():
      if not first:
        pltpu.make_async_copy(
            ones_v, dacc.at[dst_v.at[jj - 2]], dsem).wait()
      pltpu.async_copy(ones_v, dacc.at[dst_v.at[jj]], dsem, add=True)

  _gather_scatter_loop(h, src_v, dst_v, (b0, b1, b2, b3),
                       (g0, g1, g2, g3), (s0, s1, s2, s3), acc, deg)

  @pl.when(c == 0)
  def _():
    pltpu.make_async_copy(ones_v, dacc.at[dst_v.at[CPT - 2]], dsem).wait()

  @pl.when(c == 1)
  def _():
    pltpu.make_async_copy(ones_v, dacc.at[dst_v.at[CPT - 1]], dsem).wait()

  plsc.subcore_barrier()
  for k in range(RPT // ZCH):
    r = s * RPT + k * ZCH
    pltpu.sync_copy(acc.at[pl.ds(r, ZCH)], outp.at[pl.ds(r, ZCH), col])
  pltpu.sync_copy(dacc.at[pl.ds(s * RPT, RPT)], degp.at[c, pl.ds(s * RPT, RPT)])


@functools.cache
def _seg_sum_deg():
  return pl.kernel(
      _seg_sum_deg_body,
      out_type=[
          jax.ShapeDtypeStruct((ACC_R, D), jnp.float32),
          jax.ShapeDtypeStruct((NC, ACC_R, DEGW), jnp.float32),
      ],
      mesh=_mesh(),
      compiler_params=pltpu.CompilerParams(use_tc_tiling_on_sc=False),
      scratch_types=[
          pltpu.VMEM((CPT, CHUNK), jnp.int32),
          pltpu.VMEM((CPT, CHUNK), jnp.int32),
          pltpu.VMEM((CHUNK, HD), jnp.float32),
          pltpu.VMEM((CHUNK, HD), jnp.float32),
          pltpu.VMEM((CHUNK, HD), jnp.float32),
          pltpu.VMEM((CHUNK, HD), jnp.float32),
          pltpu.VMEM((CHUNK, DEGW), jnp.float32),
          pltpu.VMEM_SHARED((ACC_R, HD), jnp.float32),
          pltpu.VMEM_SHARED((ACC_R, DEGW), jnp.float32),
      ] + [pltpu.SemaphoreType.DMA] * 9,
  )


def _seg_sum_body(h, srcr, dstr, z64, outp, src_v, dst_v, b0, b1, b2, b3,
                  acc, g0, g1, g2, g3, s0, s1, s2, s3):
  c = lax.axis_index("c")
  s = lax.axis_index("s")
  for k in range(RPT // ZCH):
    pltpu.sync_copy(z64, acc.at[pl.ds(s * RPT + k * ZCH, ZCH)])
  pltpu.sync_copy(srcr.at[c, pl.ds(s * CPT, CPT)], src_v)
  pltpu.sync_copy(dstr.at[pl.ds(s * CPT, CPT)], dst_v)
  plsc.subcore_barrier()
  col = pl.ds(pl.multiple_of(c * HD, HD), HD)
  _gather_scatter_loop(h, src_v, dst_v, (b0, b1, b2, b3),
                       (g0, g1, g2, g3), (s0, s1, s2, s3), acc,
                       lambda jj, par, first: None)
  plsc.subcore_barrier()
  for k in range(RPT // ZCH):
    r = s * RPT + k * ZCH
    pltpu.sync_copy(acc.at[pl.ds(r, ZCH)], outp.at[pl.ds(r, ZCH), col])


@functools.cache
def _seg_sum():
  return pl.kernel(
      _seg_sum_body,
      out_type=jax.ShapeDtypeStruct((ACC_R, D), jnp.float32),
      mesh=_mesh(),
      compiler_params=pltpu.CompilerParams(use_tc_tiling_on_sc=False),
      scratch_types=[
          pltpu.VMEM((CPT, CHUNK), jnp.int32),
          pltpu.VMEM((CPT, CHUNK), jnp.int32),
          pltpu.VMEM((CHUNK, HD), jnp.float32),
          pltpu.VMEM((CHUNK, HD), jnp.float32),
          pltpu.VMEM((CHUNK, HD), jnp.float32),
          pltpu.VMEM((CHUNK, HD), jnp.float32),
          pltpu.VMEM_SHARED((ACC_R, HD), jnp.float32),
      ] + [pltpu.SemaphoreType.DMA] * 8,
  )


TC_B = 1000
TC_G = N_NODES // TC_B


def _embed_body(x_ref, w_ref, b_ref, o_ref):
  half = pl.program_id(0)
  h = jnp.dot(x_ref[...], w_ref[...], preferred_element_type=jnp.float32)
  h = h + b_ref[...]
  n = jnp.sqrt(jnp.sum(h * h, axis=1, keepdims=True))
  h = h / n
  o_ref[...] = jnp.where(half == 0, h[:, :HD], h[:, HD:])


def _embed(x, w, b):
  return pl.pallas_call(
      _embed_body,
      grid=(2, TC_G),
      in_specs=[
          pl.BlockSpec((TC_B, D), lambda half, i: (i, 0)),
          pl.BlockSpec((D, D), lambda half, i: (0, 0)),
          pl.BlockSpec((1, D), lambda half, i: (0, 0)),
      ],
      out_specs=pl.BlockSpec((TC_B, HD), lambda half, i: (half * TC_G + i, 0)),
      out_shape=jax.ShapeDtypeStruct((2 * N_NODES, HD), jnp.float32),
  )(x, w, b)


def _mean_block(p_ref, degp_ref):
  deg = degp_ref[0] + degp_ref[1]
  deg = jnp.sum(deg, axis=1, keepdims=True)
  deg = jnp.maximum(deg, 1.0)
  return p_ref[...] / deg


def _mid_body(p_ref, degp_ref, w_ref, b_ref, o_ref):
  half = pl.program_id(0)
  a = _mean_block(p_ref, degp_ref)
  h1 = jnp.dot(a, w_ref[...], preferred_element_type=jnp.float32) + b_ref[...]
  h1 = jnp.maximum(h1, 0.0)
  o_ref[...] = jnp.where(half == 0, h1[:, :HD], h1[:, HD:])


def _mid(p, degp, w, b):
  return pl.pallas_call(
      _mid_body,
      grid=(2, TC_G),
      in_specs=[
          pl.BlockSpec((TC_B, D), lambda half, i: (i, 0)),
          pl.BlockSpec((2, TC_B, DEGW), lambda half, i: (0, i, 0)),
          pl.BlockSpec((D, D), lambda half, i: (0, 0)),
          pl.BlockSpec((1, D), lambda half, i: (0, 0)),
      ],
      out_specs=pl.BlockSpec((TC_B, HD), lambda half, i: (half * TC_G + i, 0)),
      out_shape=jax.ShapeDtypeStruct((2 * N_NODES, HD), jnp.float32),
  )(p, degp, w, b)


def _recon_body(p_ref, degp_ref, x_ref, w_ref, b_ref, o_ref):
  a = _mean_block(p_ref, degp_ref)
  r = jnp.dot(a, w_ref[...], preferred_element_type=jnp.float32) + b_ref[...]
  r = x_ref[...] - r
  o_ref[...] = jnp.sum(r * r, axis=1).reshape(1, 1, TC_B)


def _recon(p, degp, x, w, b):
  return pl.pallas_call(
      _recon_body,
      grid=(TC_G,),
      in_specs=[
          pl.BlockSpec((TC_B, D), lambda i: (i, 0)),
          pl.BlockSpec((2, TC_B, DEGW), lambda i: (0, i, 0)),
          pl.BlockSpec((TC_B, D), lambda i: (i, 0)),
          pl.BlockSpec((D, D), lambda i: (0, 0)),
          pl.BlockSpec((1, D), lambda i: (0, 0)),
      ],
      out_specs=pl.BlockSpec((1, 1, TC_B), lambda i: (i, 0, 0)),
      out_shape=jax.ShapeDtypeStruct((TC_G, 1, TC_B), jnp.float32),
  )(p, degp, x, w, b)


def kernel(x, edge_index, W_lin, b_lin, W1, b1, W2, b2):
  n_pad = E_PAD - N_EDGES
  npc = n_pad // CHUNK
  e3 = edge_index.astype(jnp.int32).reshape(2, NCHUNK - npc, CHUNK)
  pad_src = (jnp.arange(n_pad, dtype=jnp.int32) % N_NODES).reshape(npc, CHUNK)
  pad_dst = (N_NODES
             + jnp.arange(n_pad, dtype=jnp.int32) % (ACC_R - N_NODES)
             ).reshape(npc, CHUNK)
  src = jnp.concatenate([e3[0], pad_src], axis=0)
  dst = jnp.concatenate([e3[1], pad_dst], axis=0)
  src = jnp.stack([src, src + N_NODES])
  z64 = jnp.zeros((ZCH, HD), jnp.float32)
  z16 = jnp.zeros((RPT, DEGW), jnp.float32)
  ones = jnp.ones((CHUNK, DEGW), jnp.float32)

  h = _embed(x, W_lin, b_lin.reshape(1, D))
  a1p, degp = _seg_sum_deg()(h, src, dst, z64, z16, ones)
  h1 = _mid(a1p, degp, W1, b1.reshape(1, D))
  a2p = _seg_sum()(h1, src, dst, z64)
  return _recon(a2p, degp, x, W2, b2.reshape(1, D)).reshape(N_NODES)


# speedup vs baseline: 1.1527x; 1.1527x over previous
import functools

import jax
import jax.numpy as jnp
from jax import lax
from jax.experimental import pallas as pl
from jax.experimental.pallas import tpu as pltpu
from jax.experimental.pallas import tpu_sc as plsc

N_NODES = 10000
N_EDGES = 320000
D = 128
HD = D // 2
NC, NS = 2, 16
CHUNK = 128
NCHUNK = 2560
CPT = NCHUNK // NS
E_PAD = NCHUNK * CHUNK
ACC_R = 10240
RPT = ACC_R // NS
ZCH = 128
DEGW = 16


@functools.cache
def _mesh():
  return plsc.VectorSubcoreMesh(
      core_axis_name="c", subcore_axis_name="s", num_cores=NC, num_subcores=NS)


def _gather_scatter_loop(h, src_v, dst_v, bufs, gsems, ssems, acc, deg_fn):
  def g(jj, b):
    pltpu.async_copy(h.at[src_v.at[jj]], bufs[b], gsems[b])

  def gwait(jj, b):
    pltpu.make_async_copy(h.at[src_v.at[jj]], bufs[b], gsems[b]).wait()

  def sc(jj, b):
    pltpu.async_copy(bufs[b], acc.at[dst_v.at[jj]], ssems[b], add=True)

  def scwait(jj, b):
    pltpu.make_async_copy(bufs[b], acc.at[dst_v.at[jj]], ssems[b]).wait()

  g(0, 0)
  g(1, 1)
  g(2, 2)
  gwait(0, 0)
  sc(0, 0)
  deg_fn(0, 0, True)
  g(3, 3)
  gwait(1, 1)
  sc(1, 1)
  deg_fn(1, 1, True)

  def body(i, carry):
    jj = 4 * i + 2
    for t in range(4):
      b = (2 + t) % 4
      bn = t % 4
      scwait(jj + t - 2, bn)
      g(jj + t + 2, bn)
      gwait(jj + t, b)
      sc(jj + t, b)
      deg_fn(jj + t, t % 2, False)
    return carry

  lax.fori_loop(0, (CPT - 4) // 4, body, 0)

  for jj in (CPT - 2, CPT - 1):
    gwait(jj, jj % 4)
    sc(jj, jj % 4)
    deg_fn(jj, jj % 2, False)
  for jj in range(CPT - 4, CPT):
    scwait(jj, jj % 4)


def _seg_sum_deg_body(h, srcr, dstr, z64, z16, ones, outp, degp,
                      src_v, dst_v, b0, b1, b2, b3, ones_v, acc, dacc,
                      g0, g1, g2, g3, s0, s1, s2, s3, dsem):
  c = lax.axis_index("c")
  s = lax.axis_index("s")
  for k in range(RPT // ZCH):
    pltpu.sync_copy(z64, acc.at[pl.ds(s * RPT + k * ZCH, ZCH)])
  pltpu.sync_copy(z16, dacc.at[pl.ds(s * RPT, RPT)])
  pltpu.sync_copy(ones, ones_v)
  pltpu.sync_copy(srcr.at[c, pl.ds(s * CPT, CPT)], src_v)
  pltpu.sync_copy(dstr.at[pl.ds(s * CPT, CPT)], dst_v)
  plsc.subcore_barrier()
  col = pl.ds(pl.multiple_of(c * HD, HD), HD)

  def deg(jj, par, first):
    @pl.when(c == par)
    def _():
      if not first:
        pltpu.make_async_copy(
            ones_v, dacc.at[dst_v.at[jj - 2]], dsem).wait()
      pltpu.async_copy(ones_v, dacc.at[dst_v.at[jj]], dsem, add=True)

  _gather_scatter_loop(h, src_v, dst_v, (b0, b1, b2, b3),
                       (g0, g1, g2, g3), (s0, s1, s2, s3), acc, deg)

  @pl.when(c == 0)
  def _():
    pltpu.make_async_copy(ones_v, dacc.at[dst_v.at[CPT - 2]], dsem).wait()

  @pl.when(c == 1)
  def _():
    pltpu.make_async_copy(ones_v, dacc.at[dst_v.at[CPT - 1]], dsem).wait()

  plsc.subcore_barrier()
  for k in range(RPT // ZCH):
    r = s * RPT + k * ZCH
    pltpu.sync_copy(acc.at[pl.ds(r, ZCH)], outp.at[pl.ds(r, ZCH), col])
  pltpu.sync_copy(dacc.at[pl.ds(s * RPT, RPT)], degp.at[c, pl.ds(s * RPT, RPT)])


@functools.cache
def _seg_sum_deg():
  return pl.kernel(
      _seg_sum_deg_body,
      out_type=[
          jax.ShapeDtypeStruct((ACC_R, D), jnp.float32),
          jax.ShapeDtypeStruct((NC, ACC_R, DEGW), jnp.float32),
      ],
      mesh=_mesh(),
      compiler_params=pltpu.CompilerParams(use_tc_tiling_on_sc=False),
      scratch_types=[
          pltpu.VMEM((CPT, CHUNK), jnp.int32),
          pltpu.VMEM((CPT, CHUNK), jnp.int32),
          pltpu.VMEM((CHUNK, HD), jnp.float32),
          pltpu.VMEM((CHUNK, HD), jnp.float32),
          pltpu.VMEM((CHUNK, HD), jnp.float32),
          pltpu.VMEM((CHUNK, HD), jnp.float32),
          pltpu.VMEM((CHUNK, DEGW), jnp.float32),
          pltpu.VMEM_SHARED((ACC_R, HD), jnp.float32),
          pltpu.VMEM_SHARED((ACC_R, DEGW), jnp.float32),
      ] + [pltpu.SemaphoreType.DMA] * 9,
  )


def _seg_sum_body(h, srcr, dstr, z64, outp, src_v, dst_v, b0, b1, b2, b3,
                  acc, g0, g1, g2, g3, s0, s1, s2, s3):
  c = lax.axis_index("c")
  s = lax.axis_index("s")
  for k in range(RPT // ZCH):
    pltpu.sync_copy(z64, acc.at[pl.ds(s * RPT + k * ZCH, ZCH)])
  pltpu.sync_copy(srcr.at[c, pl.ds(s * CPT, CPT)], src_v)
  pltpu.sync_copy(dstr.at[pl.ds(s * CPT, CPT)], dst_v)
  plsc.subcore_barrier()
  col = pl.ds(pl.multiple_of(c * HD, HD), HD)
  _gather_scatter_loop(h, src_v, dst_v, (b0, b1, b2, b3),
                       (g0, g1, g2, g3), (s0, s1, s2, s3), acc,
                       lambda jj, par, first: None)
  plsc.subcore_barrier()
  for k in range(RPT // ZCH):
    r = s * RPT + k * ZCH
    pltpu.sync_copy(acc.at[pl.ds(r, ZCH)], outp.at[pl.ds(r, ZCH), col])


@functools.cache
def _seg_sum():
  return pl.kernel(
      _seg_sum_body,
      out_type=jax.ShapeDtypeStruct((ACC_R, D), jnp.float32),
      mesh=_mesh(),
      compiler_params=pltpu.CompilerParams(use_tc_tiling_on_sc=False),
      scratch_types=[
          pltpu.VMEM((CPT, CHUNK), jnp.int32),
          pltpu.VMEM((CPT, CHUNK), jnp.int32),
          pltpu.VMEM((CHUNK, HD), jnp.float32),
          pltpu.VMEM((CHUNK, HD), jnp.float32),
          pltpu.VMEM((CHUNK, HD), jnp.float32),
          pltpu.VMEM((CHUNK, HD), jnp.float32),
          pltpu.VMEM_SHARED((ACC_R, HD), jnp.float32),
      ] + [pltpu.SemaphoreType.DMA] * 8,
  )


def _embed_body(x_ref, w_ref, b_ref, o_ref):
  h = jnp.dot(x_ref[...], w_ref[...], preferred_element_type=jnp.float32)
  h = h + b_ref[...]
  n = jnp.sqrt(jnp.sum(h * h, axis=1, keepdims=True))
  h = h / n
  o_ref[:N_NODES] = h[:, :HD]
  o_ref[N_NODES:] = h[:, HD:]


def _embed(x, w, b):
  return pl.pallas_call(
      _embed_body,
      out_shape=jax.ShapeDtypeStruct((2 * N_NODES, HD), jnp.float32),
  )(x, w, b)


def _mean_from_partials(p_ref, degp_ref):
  deg = degp_ref[0, :N_NODES] + degp_ref[1, :N_NODES]
  deg = jnp.sum(deg, axis=1, keepdims=True)
  deg = jnp.maximum(deg, 1.0)
  return p_ref[:N_NODES] / deg


def _mid_body(p_ref, degp_ref, w_ref, b_ref, o_ref):
  a = _mean_from_partials(p_ref, degp_ref)
  h1 = jnp.dot(a, w_ref[...], preferred_element_type=jnp.float32) + b_ref[...]
  h1 = jnp.maximum(h1, 0.0)
  o_ref[:N_NODES] = h1[:, :HD]
  o_ref[N_NODES:] = h1[:, HD:]


def _mid(p, degp, w, b):
  return pl.pallas_call(
      _mid_body,
      out_shape=jax.ShapeDtypeStruct((2 * N_NODES, HD), jnp.float32),
  )(p, degp, w, b)


def _recon_body(p_ref, degp_ref, x_ref, w_ref, b_ref, o_ref):
  a = _mean_from_partials(p_ref, degp_ref)
  r = jnp.dot(a, w_ref[...], preferred_element_type=jnp.float32) + b_ref[...]
  r = x_ref[...] - r
  o_ref[...] = jnp.sum(r * r, axis=1)


def _recon(p, degp, x, w, b):
  return pl.pallas_call(
      _recon_body,
      out_shape=jax.ShapeDtypeStruct((N_NODES,), jnp.float32),
  )(p, degp, x, w, b)


def kernel(x, edge_index, W_lin, b_lin, W1, b1, W2, b2):
  n_pad = E_PAD - N_EDGES
  pad_src = jnp.arange(n_pad, dtype=jnp.int32) % N_NODES
  pad_dst = N_NODES + jnp.arange(n_pad, dtype=jnp.int32) % (ACC_R - N_NODES)
  src = jnp.concatenate([edge_index[0].astype(jnp.int32), pad_src])
  dst = jnp.concatenate([edge_index[1].astype(jnp.int32), pad_dst])
  src = src.reshape(NCHUNK, CHUNK)
  dst = dst.reshape(NCHUNK, CHUNK)
  src = jnp.stack([src, src + N_NODES])
  z64 = jnp.zeros((ZCH, HD), jnp.float32)
  z16 = jnp.zeros((RPT, DEGW), jnp.float32)
  ones = jnp.ones((CHUNK, DEGW), jnp.float32)

  h = _embed(x, W_lin, b_lin.reshape(1, D))
  a1p, degp = _seg_sum_deg()(h, src, dst, z64, z16, ones)
  h1 = _mid(a1p, degp, W1, b1.reshape(1, D))
  a2p = _seg_sum()(h1, src, dst, z64)
  return _recon(a2p, degp, x, W2, b2.reshape(1, D))


# speedup vs baseline: 1.2020x; 1.0427x over previous
import functools

import jax
import jax.numpy as jnp
from jax import lax
from jax.experimental import pallas as pl
from jax.experimental.pallas import tpu as pltpu
from jax.experimental.pallas import tpu_sc as plsc

N_NODES = 10000
N_EDGES = 320000
D = 128
HD = D // 2
NC, NS = 2, 16
CHUNK = 128
NCHUNK = N_EDGES // CHUNK
CPT = 156
TAIL = NCHUNK - NS * CPT
ACC_R = 10240
RPT = ACC_R // NS
ZCH = 128
DEGW = 16


@functools.cache
def _mesh():
  return plsc.VectorSubcoreMesh(
      core_axis_name="c", subcore_axis_name="s", num_cores=NC, num_subcores=NS)


def _gather_scatter_loop(h, src_v, dst_v, bufs, gsems, ssems, acc, deg_fn):
  def g(jj, b):
    pltpu.async_copy(h.at[src_v.at[jj]], bufs[b], gsems[b])

  def gwait(jj, b):
    pltpu.make_async_copy(h.at[src_v.at[jj]], bufs[b], gsems[b]).wait()

  def sc(jj, b):
    pltpu.async_copy(bufs[b], acc.at[dst_v.at[jj]], ssems[b], add=True)

  def scwait(jj, b):
    pltpu.make_async_copy(bufs[b], acc.at[dst_v.at[jj]], ssems[b]).wait()

  g(0, 0)
  g(1, 1)
  g(2, 2)
  gwait(0, 0)
  sc(0, 0)
  deg_fn(0, 0, True)
  g(3, 3)
  gwait(1, 1)
  sc(1, 1)
  deg_fn(1, 1, True)

  def body(i, carry):
    jj = 4 * i + 2
    for t in range(4):
      b = (2 + t) % 4
      bn = t % 4
      scwait(jj + t - 2, bn)
      g(jj + t + 2, bn)
      gwait(jj + t, b)
      sc(jj + t, b)
      deg_fn(jj + t, t % 2, False)
    return carry

  lax.fori_loop(0, (CPT - 4) // 4, body, 0)

  for jj in (CPT - 2, CPT - 1):
    gwait(jj, jj % 4)
    sc(jj, jj % 4)
    deg_fn(jj, jj % 2, False)
  for jj in range(CPT - 4, CPT):
    scwait(jj, jj % 4)


def _stage_indices(edges, s, src_v, dst_v):
  pltpu.sync_copy(edges.at[0, pl.ds(s * CPT, CPT)], src_v.at[pl.ds(0, CPT)])
  pltpu.sync_copy(edges.at[1, pl.ds(s * CPT, CPT)], dst_v.at[pl.ds(0, CPT)])

  @pl.when(s < TAIL)
  def _():
    t = NS * CPT + s
    pltpu.sync_copy(edges.at[0, pl.ds(t, 1)], src_v.at[pl.ds(CPT, 1)])
    pltpu.sync_copy(edges.at[1, pl.ds(t, 1)], dst_v.at[pl.ds(CPT, 1)])


def _seg_sum_deg_body(h, edges, z64, z16, ones, outp, degp,
                      src_v, dst_v, b0, b1, b2, b3, ones_v, acc, dacc,
                      g0, g1, g2, g3, s0, s1, s2, s3, dsem):
  c = lax.axis_index("c")
  s = lax.axis_index("s")
  for k in range(RPT // ZCH):
    pltpu.sync_copy(z64, acc.at[pl.ds(s * RPT + k * ZCH, ZCH)])
  pltpu.sync_copy(z16, dacc.at[pl.ds(s * RPT, RPT)])
  pltpu.sync_copy(ones, ones_v)
  _stage_indices(edges, s, src_v, dst_v)
  plsc.subcore_barrier()
  col = pl.ds(pl.multiple_of(c * HD, HD), HD)
  hv = h.at[pl.ds(pl.multiple_of(c * N_NODES, N_NODES), N_NODES)]

  def deg(jj, par, first):
    @pl.when(c == par)
    def _():
      if not first:
        pltpu.make_async_copy(
            ones_v, dacc.at[dst_v.at[jj - 2]], dsem).wait()
      pltpu.async_copy(ones_v, dacc.at[dst_v.at[jj]], dsem, add=True)

  _gather_scatter_loop(hv, src_v, dst_v, (b0, b1, b2, b3),
                       (g0, g1, g2, g3), (s0, s1, s2, s3), acc, deg)

  @pl.when(c == 0)
  def _():
    pltpu.make_async_copy(ones_v, dacc.at[dst_v.at[CPT - 2]], dsem).wait()

  @pl.when(c == 1)
  def _():
    pltpu.make_async_copy(ones_v, dacc.at[dst_v.at[CPT - 1]], dsem).wait()

  @pl.when(s < TAIL)
  def _():
    pltpu.async_copy(hv.at[src_v.at[CPT]], b0, g0)
    pltpu.make_async_copy(hv.at[src_v.at[CPT]], b0, g0).wait()
    pltpu.async_copy(b0, acc.at[dst_v.at[CPT]], s0, add=True)
    pltpu.make_async_copy(b0, acc.at[dst_v.at[CPT]], s0).wait()

    @pl.when(c == s % 2)
    def _():
      pltpu.async_copy(ones_v, dacc.at[dst_v.at[CPT]], dsem, add=True)
      pltpu.make_async_copy(ones_v, dacc.at[dst_v.at[CPT]], dsem).wait()

  plsc.subcore_barrier()
  for k in range(RPT // ZCH):
    r = s * RPT + k * ZCH
    pltpu.sync_copy(acc.at[pl.ds(r, ZCH)], outp.at[pl.ds(r, ZCH), col])
  pltpu.sync_copy(dacc.at[pl.ds(s * RPT, RPT)], degp.at[c, pl.ds(s * RPT, RPT)])


@functools.cache
def _seg_sum_deg():
  return pl.kernel(
      _seg_sum_deg_body,
      out_type=[
          jax.ShapeDtypeStruct((ACC_R, D), jnp.float32),
          jax.ShapeDtypeStruct((NC, ACC_R, DEGW), jnp.float32),
      ],
      mesh=_mesh(),
      compiler_params=pltpu.CompilerParams(use_tc_tiling_on_sc=False),
      scratch_types=[
          pltpu.VMEM((CPT + 1, CHUNK), jnp.int32),
          pltpu.VMEM((CPT + 1, CHUNK), jnp.int32),
          pltpu.VMEM((CHUNK, HD), jnp.float32),
          pltpu.VMEM((CHUNK, HD), jnp.float32),
          pltpu.VMEM((CHUNK, HD), jnp.float32),
          pltpu.VMEM((CHUNK, HD), jnp.float32),
          pltpu.VMEM((CHUNK, DEGW), jnp.float32),
          pltpu.VMEM_SHARED((ACC_R, HD), jnp.float32),
          pltpu.VMEM_SHARED((ACC_R, DEGW), jnp.float32),
      ] + [pltpu.SemaphoreType.DMA] * 9,
  )


def _seg_sum_body(h, edges, z64, outp, src_v, dst_v, b0, b1, b2, b3,
                  acc, g0, g1, g2, g3, s0, s1, s2, s3):
  c = lax.axis_index("c")
  s = lax.axis_index("s")
  for k in range(RPT // ZCH):
    pltpu.sync_copy(z64, acc.at[pl.ds(s * RPT + k * ZCH, ZCH)])
  _stage_indices(edges, s, src_v, dst_v)
  plsc.subcore_barrier()
  col = pl.ds(pl.multiple_of(c * HD, HD), HD)
  hv = h.at[pl.ds(pl.multiple_of(c * N_NODES, N_NODES), N_NODES)]
  _gather_scatter_loop(hv, src_v, dst_v, (b0, b1, b2, b3),
                       (g0, g1, g2, g3), (s0, s1, s2, s3), acc,
                       lambda jj, par, first: None)

  @pl.when(s < TAIL)
  def _():
    pltpu.async_copy(hv.at[src_v.at[CPT]], b0, g0)
    pltpu.make_async_copy(hv.at[src_v.at[CPT]], b0, g0).wait()
    pltpu.async_copy(b0, acc.at[dst_v.at[CPT]], s0, add=True)
    pltpu.make_async_copy(b0, acc.at[dst_v.at[CPT]], s0).wait()

  plsc.subcore_barrier()
  for k in range(RPT // ZCH):
    r = s * RPT + k * ZCH
    pltpu.sync_copy(acc.at[pl.ds(r, ZCH)], outp.at[pl.ds(r, ZCH), col])


@functools.cache
def _seg_sum():
  return pl.kernel(
      _seg_sum_body,
      out_type=jax.ShapeDtypeStruct((ACC_R, D), jnp.float32),
      mesh=_mesh(),
      compiler_params=pltpu.CompilerParams(use_tc_tiling_on_sc=False),
      scratch_types=[
          pltpu.VMEM((CPT + 1, CHUNK), jnp.int32),
          pltpu.VMEM((CPT + 1, CHUNK), jnp.int32),
          pltpu.VMEM((CHUNK, HD), jnp.float32),
          pltpu.VMEM((CHUNK, HD), jnp.float32),
          pltpu.VMEM((CHUNK, HD), jnp.float32),
          pltpu.VMEM((CHUNK, HD), jnp.float32),
          pltpu.VMEM_SHARED((ACC_R, HD), jnp.float32),
      ] + [pltpu.SemaphoreType.DMA] * 8,
  )


def _embed_body(x_ref, w_ref, b_ref, o_ref):
  h = jnp.dot(x_ref[...], w_ref[...], preferred_element_type=jnp.float32)
  h = h + b_ref[...]
  n = jnp.sqrt(jnp.sum(h * h, axis=1, keepdims=True))
  h = h / n
  o_ref[:N_NODES] = h[:, :HD]
  o_ref[N_NODES:] = h[:, HD:]


def _embed(x, w, b):
  return pl.pallas_call(
      _embed_body,
      out_shape=jax.ShapeDtypeStruct((2 * N_NODES, HD), jnp.float32),
  )(x, w, b)


def _mean_from_partials(p_ref, degp_ref):
  deg = degp_ref[0, :N_NODES] + degp_ref[1, :N_NODES]
  deg = jnp.sum(deg, axis=1, keepdims=True)
  deg = jnp.maximum(deg, 1.0)
  return p_ref[:N_NODES] / deg


def _mid_body(p_ref, degp_ref, w_ref, b_ref, o_ref):
  a = _mean_from_partials(p_ref, degp_ref)
  h1 = jnp.dot(a, w_ref[...], preferred_element_type=jnp.float32) + b_ref[...]
  h1 = jnp.maximum(h1, 0.0)
  o_ref[:N_NODES] = h1[:, :HD]
  o_ref[N_NODES:] = h1[:, HD:]


def _mid(p, degp, w, b):
  return pl.pallas_call(
      _mid_body,
      out_shape=jax.ShapeDtypeStruct((2 * N_NODES, HD), jnp.float32),
  )(p, degp, w, b)


def _recon_body(p_ref, degp_ref, x_ref, w_ref, b_ref, o_ref):
  a = _mean_from_partials(p_ref, degp_ref)
  r = jnp.dot(a, w_ref[...], preferred_element_type=jnp.float32) + b_ref[...]
  r = x_ref[...] - r
  o_ref[...] = jnp.sum(r * r, axis=1)


def _recon(p, degp, x, w, b):
  return pl.pallas_call(
      _recon_body,
      out_shape=jax.ShapeDtypeStruct((N_NODES,), jnp.float32),
  )(p, degp, x, w, b)


def kernel(x, edge_index, W_lin, b_lin, W1, b1, W2, b2):
  edges = edge_index.astype(jnp.int32).reshape(2, NCHUNK, CHUNK)
  z64 = jnp.zeros((ZCH, HD), jnp.float32)
  z16 = jnp.zeros((RPT, DEGW), jnp.float32)
  ones = jnp.ones((CHUNK, DEGW), jnp.float32)

  h = _embed(x, W_lin, b_lin.reshape(1, D))
  a1p, degp = _seg_sum_deg()(h, edges, z64, z16, ones)
  h1 = _mid(a1p, degp, W1, b1.reshape(1, D))
  a2p = _seg_sum()(h1, edges, z64)
  return _recon(a2p, degp, x, W2, b2.reshape(1, D))


# speedup vs baseline: 1.2186x; 1.0139x over previous
import functools

import jax
import jax.numpy as jnp
from jax import lax
from jax.experimental import pallas as pl
from jax.experimental.pallas import tpu as pltpu
from jax.experimental.pallas import tpu_sc as plsc

N_NODES = 10000
N_EDGES = 320000
D = 128
HD = D // 2
NC, NS = 2, 16
CHUNK = 128
NCHUNK = N_EDGES // CHUNK
CPT = 156
TAIL = NCHUNK - NS * CPT
ACC_R = 10240
RPT = ACC_R // NS
ZCH = 128
DEGW = 16


@functools.cache
def _mesh():
  return plsc.VectorSubcoreMesh(
      core_axis_name="c", subcore_axis_name="s", num_cores=NC, num_subcores=NS)


def _gather_scatter_loop(h, src_v, dst_v, bufs, gsems, ssems, acc, deg_fn):
  def g(jj, b):
    pltpu.async_copy(h.at[src_v.at[jj]], bufs[b], gsems[b])

  def gwait(jj, b):
    pltpu.make_async_copy(h.at[src_v.at[jj]], bufs[b], gsems[b]).wait()

  def sc(jj, b):
    pltpu.async_copy(bufs[b], acc.at[dst_v.at[jj]], ssems[b], add=True)

  def scwait(jj, b):
    pltpu.make_async_copy(bufs[b], acc.at[dst_v.at[jj]], ssems[b]).wait()

  g(0, 0)
  g(1, 1)
  g(2, 2)
  gwait(0, 0)
  sc(0, 0)
  deg_fn(0, 0, True)
  g(3, 3)
  gwait(1, 1)
  sc(1, 1)
  deg_fn(1, 1, True)

  def body(i, carry):
    jj = 4 * i + 2
    for t in range(4):
      b = (2 + t) % 4
      bn = t % 4
      scwait(jj + t - 2, bn)
      g(jj + t + 2, bn)
      gwait(jj + t, b)
      sc(jj + t, b)
      deg_fn(jj + t, t % 2, False)
    return carry

  lax.fori_loop(0, (CPT - 4) // 4, body, 0)

  for jj in (CPT - 2, CPT - 1):
    gwait(jj, jj % 4)
    sc(jj, jj % 4)
    deg_fn(jj, jj % 2, False)
  for jj in range(CPT - 4, CPT):
    scwait(jj, jj % 4)


def _stage_indices(edges, s, src_v, dst_v, sem):
  pltpu.async_copy(edges.at[0, pl.ds(s * CPT, CPT)], src_v.at[pl.ds(0, CPT)],
                   sem)
  pltpu.async_copy(edges.at[1, pl.ds(s * CPT, CPT)], dst_v.at[pl.ds(0, CPT)],
                   sem)

  @pl.when(s < TAIL)
  def _():
    t = NS * CPT + s
    pltpu.async_copy(edges.at[0, pl.ds(t, 1)], src_v.at[pl.ds(CPT, 1)], sem)
    pltpu.async_copy(edges.at[1, pl.ds(t, 1)], dst_v.at[pl.ds(CPT, 1)], sem)


def _stage_indices_wait(edges, s, src_v, dst_v, sem):
  pltpu.make_async_copy(edges.at[0, pl.ds(s * CPT, CPT)],
                        src_v.at[pl.ds(0, CPT)], sem).wait()
  pltpu.make_async_copy(edges.at[1, pl.ds(s * CPT, CPT)],
                        dst_v.at[pl.ds(0, CPT)], sem).wait()

  @pl.when(s < TAIL)
  def _():
    t = NS * CPT + s
    pltpu.make_async_copy(edges.at[0, pl.ds(t, 1)],
                          src_v.at[pl.ds(CPT, 1)], sem).wait()
    pltpu.make_async_copy(edges.at[1, pl.ds(t, 1)],
                          dst_v.at[pl.ds(CPT, 1)], sem).wait()


def _seg_sum_deg_body(h, edges, z64, z16, ones, outp, degp,
                      src_v, dst_v, b0, b1, b2, b3, ones_v, acc, dacc,
                      g0, g1, g2, g3, s0, s1, s2, s3, dsem):
  c = lax.axis_index("c")
  s = lax.axis_index("s")
  for k in range(RPT // ZCH):
    pltpu.async_copy(z64, acc.at[pl.ds(s * RPT + k * ZCH, ZCH)], g0)
  pltpu.async_copy(z16, dacc.at[pl.ds(s * RPT, RPT)], g1)
  pltpu.async_copy(ones, ones_v, g2)
  _stage_indices(edges, s, src_v, dst_v, g3)
  for k in range(RPT // ZCH):
    pltpu.make_async_copy(z64, acc.at[pl.ds(s * RPT + k * ZCH, ZCH)], g0).wait()
  pltpu.make_async_copy(z16, dacc.at[pl.ds(s * RPT, RPT)], g1).wait()
  pltpu.make_async_copy(ones, ones_v, g2).wait()
  _stage_indices_wait(edges, s, src_v, dst_v, g3)
  plsc.subcore_barrier()
  col = pl.ds(pl.multiple_of(c * HD, HD), HD)
  hv = h.at[pl.ds(pl.multiple_of(c * N_NODES, N_NODES), N_NODES)]

  def deg(jj, par, first):
    @pl.when(c == par)
    def _():
      if not first:
        pltpu.make_async_copy(
            ones_v, dacc.at[dst_v.at[jj - 2]], dsem).wait()
      pltpu.async_copy(ones_v, dacc.at[dst_v.at[jj]], dsem, add=True)

  _gather_scatter_loop(hv, src_v, dst_v, (b0, b1, b2, b3),
                       (g0, g1, g2, g3), (s0, s1, s2, s3), acc, deg)

  @pl.when(c == 0)
  def _():
    pltpu.make_async_copy(ones_v, dacc.at[dst_v.at[CPT - 2]], dsem).wait()

  @pl.when(c == 1)
  def _():
    pltpu.make_async_copy(ones_v, dacc.at[dst_v.at[CPT - 1]], dsem).wait()

  @pl.when(s < TAIL)
  def _():
    pltpu.async_copy(hv.at[src_v.at[CPT]], b0, g0)
    pltpu.make_async_copy(hv.at[src_v.at[CPT]], b0, g0).wait()
    pltpu.async_copy(b0, acc.at[dst_v.at[CPT]], s0, add=True)
    pltpu.make_async_copy(b0, acc.at[dst_v.at[CPT]], s0).wait()

    @pl.when(c == s % 2)
    def _():
      pltpu.async_copy(ones_v, dacc.at[dst_v.at[CPT]], dsem, add=True)
      pltpu.make_async_copy(ones_v, dacc.at[dst_v.at[CPT]], dsem).wait()

  plsc.subcore_barrier()
  for k in range(RPT // ZCH):
    r = s * RPT + k * ZCH
    pltpu.async_copy(acc.at[pl.ds(r, ZCH)], outp.at[pl.ds(r, ZCH), col], s0)
  pltpu.async_copy(dacc.at[pl.ds(s * RPT, RPT)],
                   degp.at[c, pl.ds(s * RPT, RPT)], s1)
  for k in range(RPT // ZCH):
    r = s * RPT + k * ZCH
    pltpu.make_async_copy(
        acc.at[pl.ds(r, ZCH)], outp.at[pl.ds(r, ZCH), col], s0).wait()
  pltpu.make_async_copy(dacc.at[pl.ds(s * RPT, RPT)],
                        degp.at[c, pl.ds(s * RPT, RPT)], s1).wait()


@functools.cache
def _seg_sum_deg():
  return pl.kernel(
      _seg_sum_deg_body,
      out_type=[
          jax.ShapeDtypeStruct((ACC_R, D), jnp.float32),
          jax.ShapeDtypeStruct((NC, ACC_R, DEGW), jnp.float32),
      ],
      mesh=_mesh(),
      compiler_params=pltpu.CompilerParams(use_tc_tiling_on_sc=False),
      scratch_types=[
          pltpu.VMEM((CPT + 1, CHUNK), jnp.int32),
          pltpu.VMEM((CPT + 1, CHUNK), jnp.int32),
          pltpu.VMEM((CHUNK, HD), jnp.float32),
          pltpu.VMEM((CHUNK, HD), jnp.float32),
          pltpu.VMEM((CHUNK, HD), jnp.float32),
          pltpu.VMEM((CHUNK, HD), jnp.float32),
          pltpu.VMEM((CHUNK, DEGW), jnp.float32),
          pltpu.VMEM_SHARED((ACC_R, HD), jnp.float32),
          pltpu.VMEM_SHARED((ACC_R, DEGW), jnp.float32),
      ] + [pltpu.SemaphoreType.DMA] * 9,
  )


def _seg_sum_body(h, edges, z64, outp, src_v, dst_v, b0, b1, b2, b3,
                  acc, g0, g1, g2, g3, s0, s1, s2, s3):
  c = lax.axis_index("c")
  s = lax.axis_index("s")
  for k in range(RPT // ZCH):
    pltpu.async_copy(z64, acc.at[pl.ds(s * RPT + k * ZCH, ZCH)], g0)
  _stage_indices(edges, s, src_v, dst_v, g1)
  for k in range(RPT // ZCH):
    pltpu.make_async_copy(z64, acc.at[pl.ds(s * RPT + k * ZCH, ZCH)], g0).wait()
  _stage_indices_wait(edges, s, src_v, dst_v, g1)
  plsc.subcore_barrier()
  col = pl.ds(pl.multiple_of(c * HD, HD), HD)
  hv = h.at[pl.ds(pl.multiple_of(c * N_NODES, N_NODES), N_NODES)]
  _gather_scatter_loop(hv, src_v, dst_v, (b0, b1, b2, b3),
                       (g0, g1, g2, g3), (s0, s1, s2, s3), acc,
                       lambda jj, par, first: None)

  @pl.when(s < TAIL)
  def _():
    pltpu.async_copy(hv.at[src_v.at[CPT]], b0, g0)
    pltpu.make_async_copy(hv.at[src_v.at[CPT]], b0, g0).wait()
    pltpu.async_copy(b0, acc.at[dst_v.at[CPT]], s0, add=True)
    pltpu.make_async_copy(b0, acc.at[dst_v.at[CPT]], s0).wait()

  plsc.subcore_barrier()
  for k in range(RPT // ZCH):
    r = s * RPT + k * ZCH
    pltpu.async_copy(acc.at[pl.ds(r, ZCH)], outp.at[pl.ds(r, ZCH), col], s0)
  for k in range(RPT // ZCH):
    r = s * RPT + k * ZCH
    pltpu.make_async_copy(
        acc.at[pl.ds(r, ZCH)], outp.at[pl.ds(r, ZCH), col], s0).wait()


@functools.cache
def _seg_sum():
  return pl.kernel(
      _seg_sum_body,
      out_type=jax.ShapeDtypeStruct((ACC_R, D), jnp.float32),
      mesh=_mesh(),
      compiler_params=pltpu.CompilerParams(use_tc_tiling_on_sc=False),
      scratch_types=[
          pltpu.VMEM((CPT + 1, CHUNK), jnp.int32),
          pltpu.VMEM((CPT + 1, CHUNK), jnp.int32),
          pltpu.VMEM((CHUNK, HD), jnp.float32),
          pltpu.VMEM((CHUNK, HD), jnp.float32),
          pltpu.VMEM((CHUNK, HD), jnp.float32),
          pltpu.VMEM((CHUNK, HD), jnp.float32),
          pltpu.VMEM_SHARED((ACC_R, HD), jnp.float32),
      ] + [pltpu.SemaphoreType.DMA] * 8,
  )


def _embed_body(x_ref, w_ref, b_ref, o_ref):
  h = jnp.dot(x_ref[...], w_ref[...], preferred_element_type=jnp.float32)
  h = h + b_ref[...]
  n = jnp.sqrt(jnp.sum(h * h, axis=1, keepdims=True))
  h = h / n
  o_ref[:N_NODES] = h[:, :HD]
  o_ref[N_NODES:] = h[:, HD:]


def _embed(x, w, b):
  return pl.pallas_call(
      _embed_body,
      out_shape=jax.ShapeDtypeStruct((2 * N_NODES, HD), jnp.float32),
  )(x, w, b)


def _mean_from_partials(p_ref, degp_ref):
  deg = degp_ref[0, :N_NODES] + degp_ref[1, :N_NODES]
  deg = jnp.sum(deg, axis=1, keepdims=True)
  deg = jnp.maximum(deg, 1.0)
  return p_ref[:N_NODES] / deg


def _mid_body(p_ref, degp_ref, w_ref, b_ref, o_ref):
  a = _mean_from_partials(p_ref, degp_ref)
  h1 = jnp.dot(a, w_ref[...], preferred_element_type=jnp.float32) + b_ref[...]
  h1 = jnp.maximum(h1, 0.0)
  o_ref[:N_NODES] = h1[:, :HD]
  o_ref[N_NODES:] = h1[:, HD:]


def _mid(p, degp, w, b):
  return pl.pallas_call(
      _mid_body,
      out_shape=jax.ShapeDtypeStruct((2 * N_NODES, HD), jnp.float32),
  )(p, degp, w, b)


def _recon_body(p_ref, degp_ref, x_ref, w_ref, b_ref, o_ref):
  a = _mean_from_partials(p_ref, degp_ref)
  r = jnp.dot(a, w_ref[...], preferred_element_type=jnp.float32) + b_ref[...]
  r = x_ref[...] - r
  o_ref[...] = jnp.sum(r * r, axis=1)


def _recon(p, degp, x, w, b):
  return pl.pallas_call(
      _recon_body,
      out_shape=jax.ShapeDtypeStruct((N_NODES,), jnp.float32),
  )(p, degp, x, w, b)


def kernel(x, edge_index, W_lin, b_lin, W1, b1, W2, b2):
  edges = edge_index.astype(jnp.int32).reshape(2, NCHUNK, CHUNK)
  z64 = jnp.zeros((ZCH, HD), jnp.float32)
  z16 = jnp.zeros((RPT, DEGW), jnp.float32)
  ones = jnp.ones((CHUNK, DEGW), jnp.float32)

  h = _embed(x, W_lin, b_lin.reshape(1, D))
  a1p, degp = _seg_sum_deg()(h, edges, z64, z16, ones)
  h1 = _mid(a1p, degp, W1, b1.reshape(1, D))
  a2p = _seg_sum()(h1, edges, z64)
  return _recon(a2p, degp, x, W2, b2.reshape(1, D))
